# baseline (speedup 1.0000x reference)
import functools

import jax
import jax.numpy as jnp
from jax import lax
from jax.experimental import pallas as pl
from jax.experimental.pallas import tpu as pltpu
from jax.experimental.pallas import tpu_sc as plsc

_NC = 2
_NS = 16
_NW = _NC * _NS
_C = 16
_NBUF = 6
_LANES = 16


def _add_chunk(buf, ids_v, tbl_v, u, e):
    kmax = e // _LANES

    def _row(j):
        tb = ids_v[pl.ds(u * _C + j, _LANES)][0] * e

        def _lane(k):
            off = k * _LANES
            buf[j, pl.ds(off, _LANES)] = (
                buf[j, pl.ds(off, _LANES)] + tbl_v[pl.ds(tb + off, _LANES)])

        plsc.parallel_loop(0, kmax, unroll=16)(_lane)

    plsc.parallel_loop(0, _C)(_row)


def _sc_body(x_hbm, ids_hbm, tbl_hbm, out_hbm, ids_v, tbl_v, *bufs_and_sems):
    n_chunks = ids_hbm.shape[1] // _C
    e = x_hbm.shape[1]
    wid = lax.axis_index("s") * _NC + lax.axis_index("c")
    rpw = n_chunks * _C
    base = wid * rpw
    bufs = list(bufs_and_sems[:_NBUF])
    sins = list(bufs_and_sems[_NBUF:2 * _NBUF])
    souts = list(bufs_and_sems[2 * _NBUF:3 * _NBUF])
    s_tbl = bufs_and_sems[3 * _NBUF]
    s_ids = bufs_and_sems[3 * _NBUF + 1]

    tbl_cp = pltpu.async_copy(tbl_hbm, tbl_v, s_tbl)
    ids_cp = pltpu.async_copy(
        ids_hbm.at[wid], ids_v.at[pl.ds(0, ids_hbm.shape[1])], s_ids)

    lead = 4
    in_cp = [None] * n_chunks
    out_cp = [None] * n_chunks
    for t in range(n_chunks + lead):
        if t < n_chunks:
            bi = t % _NBUF
            if t >= _NBUF:
                out_cp[t - _NBUF].wait()
            in_cp[t] = pltpu.async_copy(
                x_hbm.at[pl.ds(base + t * _C, _C)], bufs[bi], sins[bi])
        if t == lead:
            tbl_cp.wait()
            ids_cp.wait()
        if t >= lead:
            u = t - lead
            bu = u % _NBUF
            in_cp[u].wait()
            _add_chunk(bufs[bu], ids_v, tbl_v, u, e)
            out_cp[u] = pltpu.async_copy(
                bufs[bu], out_hbm.at[pl.ds(base + u * _C, _C)], souts[bu])
    for u in range(max(0, n_chunks - _NBUF), n_chunks):
        out_cp[u].wait()


def kernel(input_tensor, token_type_ids, token_type_table):
    b, s, e = input_tensor.shape
    n = b * s
    rpw = n // _NW
    x = input_tensor.reshape(n, e)
    ids = token_type_ids.reshape(_NW, rpw).astype(jnp.int32)
    tbl = token_type_table.reshape(-1)

    mesh = plsc.VectorSubcoreMesh(core_axis_name="c", subcore_axis_name="s")
    sc_k = functools.partial(
        pl.kernel,
        out_type=jax.ShapeDtypeStruct((n, e), jnp.float32),
        mesh=mesh,
        scratch_types=[
            pltpu.VMEM((rpw + _LANES,), jnp.int32),
            pltpu.VMEM((token_type_table.size,), jnp.float32),
        ] + [pltpu.VMEM((_C, e), jnp.float32)] * _NBUF
          + [pltpu.SemaphoreType.DMA] * (2 * _NBUF + 2),
    )(_sc_body)
    out = sc_k(x, ids, tbl)
    return out.reshape(b, s, e)


# speedup vs baseline: 1.0114x; 1.0114x over previous
import functools

import jax
import jax.numpy as jnp
from jax import lax
from jax.experimental import pallas as pl
from jax.experimental.pallas import tpu as pltpu
from jax.experimental.pallas import tpu_sc as plsc

_NC = 2
_NS = 16
_NW = _NC * _NS
_C = 16
_NBUF = 6
_LANES = 16


def _add_chunk(buf, ids_v, tbl_v, u, e):
    kmax = e // _LANES

    def _row(j):
        tb = ids_v[pl.ds(u * _C + j, _LANES)][0] * e

        def _lane(k):
            off = k * _LANES
            buf[j, pl.ds(off, _LANES)] = (
                buf[j, pl.ds(off, _LANES)] + tbl_v[pl.ds(tb + off, _LANES)])

        plsc.parallel_loop(0, kmax, unroll=16)(_lane)

    plsc.parallel_loop(0, _C)(_row)


def _sc_body(x_hbm, ids_hbm, tbl_hbm, out_hbm, ids_v, tbl_v, *bufs_and_sems):
    n_chunks = ids_hbm.shape[1] // _C
    e = x_hbm.shape[1]
    wid = lax.axis_index("s") * _NC + lax.axis_index("c")
    rpw = n_chunks * _C
    base = wid * rpw
    bufs = list(bufs_and_sems[:_NBUF])
    sins = list(bufs_and_sems[_NBUF:2 * _NBUF])
    souts = list(bufs_and_sems[2 * _NBUF:3 * _NBUF])
    s_tbl = bufs_and_sems[3 * _NBUF]
    s_ids = bufs_and_sems[3 * _NBUF + 1]

    tbl_cp = pltpu.async_copy(tbl_hbm, tbl_v, s_tbl)
    ids_cp = pltpu.async_copy(
        ids_hbm.at[wid], ids_v.at[pl.ds(0, ids_hbm.shape[1])], s_ids)

    lead = 3
    in_cp = [None] * n_chunks
    out_cp = [None] * n_chunks
    for t in range(n_chunks + lead):
        if t < n_chunks:
            bi = t % _NBUF
            if t >= _NBUF:
                out_cp[t - _NBUF].wait()
            in_cp[t] = pltpu.async_copy(
                x_hbm.at[pl.ds(base + t * _C, _C)], bufs[bi], sins[bi])
        if t == lead:
            tbl_cp.wait()
            ids_cp.wait()
        if t >= lead:
            u = t - lead
            bu = u % _NBUF
            in_cp[u].wait()
            _add_chunk(bufs[bu], ids_v, tbl_v, u, e)
            out_cp[u] = pltpu.async_copy(
                bufs[bu], out_hbm.at[pl.ds(base + u * _C, _C)], souts[bu])
    for u in range(max(0, n_chunks - _NBUF), n_chunks):
        out_cp[u].wait()


def kernel(input_tensor, token_type_ids, token_type_table):
    b, s, e = input_tensor.shape
    n = b * s
    rpw = n // _NW
    x = input_tensor.reshape(n, e)
    ids = token_type_ids.reshape(_NW, rpw).astype(jnp.int32)
    tbl = token_type_table.reshape(-1)

    mesh = plsc.VectorSubcoreMesh(core_axis_name="c", subcore_axis_name="s")
    sc_k = functools.partial(
        pl.kernel,
        out_type=jax.ShapeDtypeStruct((n, e), jnp.float32),
        mesh=mesh,
        scratch_types=[
            pltpu.VMEM((rpw + _LANES,), jnp.int32),
            pltpu.VMEM((token_type_table.size,), jnp.float32),
        ] + [pltpu.VMEM((_C, e), jnp.float32)] * _NBUF
          + [pltpu.SemaphoreType.DMA] * (2 * _NBUF + 2),
    )(_sc_body)
    out = sc_k(x, ids, tbl)
    return out.reshape(b, s, e)
